# Initial kernel scaffold; baseline (speedup 1.0000x reference)
import functools

import jax
import jax.numpy as jnp
from jax import lax
from jax.experimental import pallas as pl
from jax.experimental.pallas import tpu as pltpu
from jax.experimental.pallas import tpu_sc as plsc

N = 10000
E = 160000
F_IN = 256
H = 512
G = 64

NP = 10240
EP = 163840
PAD_SRC = N
PAD_DST = N + 16
NC = 2
NS = 16
ROWS_PT = NP // NS
BATCH = 128

f32 = jnp.float32
i32 = jnp.int32

_SC_MESH = dict(mesh=plsc.VectorSubcoreMesh(core_axis_name="c",
                                            subcore_axis_name="s"))


def _sc_deg(dst2d, ones_np, zeros_np):
    nb = EP // (NC * NS) // BATCH

    @functools.partial(
        pl.kernel,
        out_type=(jax.ShapeDtypeStruct((NP,), f32),
                  jax.ShapeDtypeStruct((NP,), f32)),
        scratch_types=[
            pltpu.VMEM((nb, BATCH), i32),
            pltpu.VMEM((BATCH,), f32),
            pltpu.VMEM_SHARED((NP,), f32),
        ],
        **_SC_MESH,
    )
    def k(dst_hbm, ones_hbm, zeros_hbm, out0, out1, idx_v, ones_v, acc):
        c = lax.axis_index("c")
        s = lax.axis_index("s")
        wid = c * NS + s

        @pl.when(c == 0)
        def _():
            pltpu.sync_copy(ones_hbm.at[pl.ds(s * ROWS_PT, ROWS_PT)],
                            acc.at[pl.ds(s * ROWS_PT, ROWS_PT)])

        @pl.when(c == 1)
        def _():
            pltpu.sync_copy(zeros_hbm.at[pl.ds(s * ROWS_PT, ROWS_PT)],
                            acc.at[pl.ds(s * ROWS_PT, ROWS_PT)])

        pltpu.sync_copy(ones_hbm.at[pl.ds(0, BATCH)], ones_v)
        pltpu.sync_copy(dst_hbm.at[pl.ds(wid * nb, nb)], idx_v)
        plsc.subcore_barrier()

        def body(j, carry):
            pltpu.sync_copy(ones_v, acc.at[idx_v.at[j]], add=True)
            return carry

        lax.fori_loop(0, nb, body, 0)
        plsc.subcore_barrier()

        @pl.when(c == 0)
        def _():
            pltpu.sync_copy(acc.at[pl.ds(s * ROWS_PT, ROWS_PT)],
                            out0.at[pl.ds(s * ROWS_PT, ROWS_PT)])

        @pl.when(c == 1)
        def _():
            pltpu.sync_copy(acc.at[pl.ds(s * ROWS_PT, ROWS_PT)],
                            out1.at[pl.ds(s * ROWS_PT, ROWS_PT)])

    return k(dst2d, ones_np, zeros_np)


def _make_sc_agg(cpc):
    nb = EP // NS // BATCH
    nchunks = NC * cpc

    @functools.partial(
        pl.kernel,
        out_type=tuple(jax.ShapeDtypeStruct((NP, 128), f32)
                       for _ in range(nchunks)),
        scratch_types=[
            pltpu.VMEM((nb, BATCH), i32),
            pltpu.VMEM((nb, BATCH), i32),
            pltpu.VMEM((BATCH, 128), f32),
            pltpu.VMEM_SHARED((NP, 128), f32),
            pltpu.SemaphoreType.DMA,
        ],
        **_SC_MESH,
    )
    def k(*args):
        xp = args[:nchunks]
        src2d, dst2d = args[nchunks], args[nchunks + 1]
        outs = args[nchunks + 2:2 * nchunks + 2]
        src_v, dst_v, rows, acc, sem = args[2 * nchunks + 2:]
        c = lax.axis_index("c")
        s = lax.axis_index("s")

        pltpu.sync_copy(src2d.at[pl.ds(s * nb, nb)], src_v)
        pltpu.sync_copy(dst2d.at[pl.ds(s * nb, nb)], dst_v)

        def do_chunk(xp_ref, out_ref):
            pltpu.sync_copy(xp_ref.at[pl.ds(s * ROWS_PT, ROWS_PT)],
                            acc.at[pl.ds(s * ROWS_PT, ROWS_PT)])
            plsc.subcore_barrier()

            def body(j, carry):
                pltpu.async_copy(xp_ref.at[src_v.at[j]], rows, sem).wait()
                pltpu.sync_copy(rows, acc.at[dst_v.at[j]], add=True)
                return carry

            lax.fori_loop(0, nb, body, 0)
            plsc.subcore_barrier()
            pltpu.sync_copy(acc.at[pl.ds(s * ROWS_PT, ROWS_PT)],
                            out_ref.at[pl.ds(s * ROWS_PT, ROWS_PT)])

        for cc in range(NC):
            @pl.when(c == cc)
            def _(cc=cc):
                for q in range(cpc):
                    do_chunk(xp[cc * cpc + q], outs[cc * cpc + q])

    return k


_sc_agg2 = _make_sc_agg(1)
_sc_agg4 = _make_sc_agg(2)


def _sc_agg_scalar(z_hbm_in, zeros_np, src2d, dst2d):
    nb = EP // (NC * NS) // BATCH

    @functools.partial(
        pl.kernel,
        out_type=(jax.ShapeDtypeStruct((NP,), f32),
                  jax.ShapeDtypeStruct((NP,), f32)),
        scratch_types=[
            pltpu.VMEM((nb, BATCH), i32),
            pltpu.VMEM((nb, BATCH), i32),
            pltpu.VMEM((BATCH,), f32),
            pltpu.VMEM_SHARED((NP,), f32),
            pltpu.VMEM_SHARED((NP,), f32),
            pltpu.SemaphoreType.DMA,
        ],
        **_SC_MESH,
    )
    def k(z_hbm, zeros_hbm, src_hbm, dst_hbm, out0, out1,
          src_v, dst_v, vals, z_sp, acc, sem):
        c = lax.axis_index("c")
        s = lax.axis_index("s")
        wid = c * NS + s

        pltpu.sync_copy(z_hbm.at[pl.ds(s * ROWS_PT, ROWS_PT)],
                        z_sp.at[pl.ds(s * ROWS_PT, ROWS_PT)])

        @pl.when(c == 0)
        def _():
            pltpu.sync_copy(z_hbm.at[pl.ds(s * ROWS_PT, ROWS_PT)],
                            acc.at[pl.ds(s * ROWS_PT, ROWS_PT)])

        @pl.when(c == 1)
        def _():
            pltpu.sync_copy(zeros_hbm.at[pl.ds(s * ROWS_PT, ROWS_PT)],
                            acc.at[pl.ds(s * ROWS_PT, ROWS_PT)])

        pltpu.sync_copy(src_hbm.at[pl.ds(wid * nb, nb)], src_v)
        pltpu.sync_copy(dst_hbm.at[pl.ds(wid * nb, nb)], dst_v)
        plsc.subcore_barrier()

        def body(j, carry):
            pltpu.async_copy(z_sp.at[src_v.at[j]], vals, sem).wait()
            pltpu.sync_copy(vals, acc.at[dst_v.at[j]], add=True)
            return carry

        lax.fori_loop(0, nb, body, 0)
        plsc.subcore_barrier()

        @pl.when(c == 0)
        def _():
            pltpu.sync_copy(acc.at[pl.ds(s * ROWS_PT, ROWS_PT)],
                            out0.at[pl.ds(s * ROWS_PT, ROWS_PT)])

        @pl.when(c == 1)
        def _():
            pltpu.sync_copy(acc.at[pl.ds(s * ROWS_PT, ROWS_PT)],
                            out1.at[pl.ds(s * ROWS_PT, ROWS_PT)])

    return k(z_hbm_in, zeros_np, src2d, dst2d)


_BR = 1024


def _tc_prep_body(deg0, deg1, x, dinv, dinv_row, xp0, xp1):
    i = pl.program_id(0)
    rows = i * _BR + lax.broadcasted_iota(i32, (_BR, 1), 0)
    deg = deg0[...] + deg1[...]
    dv = jnp.where(rows < N, lax.rsqrt(deg), 0.0)
    dinv[...] = dv
    dinv_row[...] = dv.reshape(1, _BR)
    xp0[...] = x[:, :128] * dv
    xp1[...] = x[:, 128:] * dv


def _tc_prep(deg0, deg1, x):
    return pl.pallas_call(
        _tc_prep_body,
        grid=(NP // _BR,),
        in_specs=[
            pl.BlockSpec((_BR, 1), lambda i: (i, 0)),
            pl.BlockSpec((_BR, 1), lambda i: (i, 0)),
            pl.BlockSpec((_BR, F_IN), lambda i: (i, 0)),
        ],
        out_specs=[
            pl.BlockSpec((_BR, 1), lambda i: (i, 0)),
            pl.BlockSpec((1, _BR), lambda i: (0, i)),
            pl.BlockSpec((_BR, 128), lambda i: (i, 0)),
            pl.BlockSpec((_BR, 128), lambda i: (i, 0)),
        ],
        out_shape=[
            jax.ShapeDtypeStruct((NP, 1), f32),
            jax.ShapeDtypeStruct((1, NP), f32),
            jax.ShapeDtypeStruct((NP, 128), f32),
            jax.ShapeDtypeStruct((NP, 128), f32),
        ],
    )(deg0, deg1, x)


def _tc_layer1_body(s10, s11, dinv, W1, b1, *outs):
    dv = dinv[...]
    h = (jnp.dot(s10[...] * dv, W1[:128, :], preferred_element_type=f32)
         + jnp.dot(s11[...] * dv, W1[128:, :], preferred_element_type=f32)
         + b1[...])
    h = jnp.maximum(h, 0.0) * dv
    for q in range(4):
        outs[q][...] = h[:, q * 128:(q + 1) * 128]


def _tc_layer1(s10, s11, dinv, W1, b1):
    return pl.pallas_call(
        _tc_layer1_body,
        grid=(NP // _BR,),
        in_specs=[
            pl.BlockSpec((_BR, 128), lambda i: (i, 0)),
            pl.BlockSpec((_BR, 128), lambda i: (i, 0)),
            pl.BlockSpec((_BR, 1), lambda i: (i, 0)),
            pl.BlockSpec((F_IN, H), lambda i: (0, 0)),
            pl.BlockSpec((1, H), lambda i: (0, 0)),
        ],
        out_specs=[pl.BlockSpec((_BR, 128), lambda i: (i, 0))
                   for _ in range(4)],
        out_shape=[jax.ShapeDtypeStruct((NP, 128), f32) for _ in range(4)],
    )(s10, s11, dinv, W1, b1)


def _tc_layer2_body(s20, s21, s22, s23, dinv, W2, b2, W3, Wp, z):
    dv = dinv[...]
    ss = (s20, s21, s22, s23)
    h = b2[...]
    for q in range(4):
        h = h + jnp.dot(ss[q][...] * dv, W2[q * 128:(q + 1) * 128, :],
                        preferred_element_type=f32)
    h = jnp.maximum(h, 0.0)
    w3p = jnp.dot(W3[...], Wp[...], preferred_element_type=f32)
    z[...] = jnp.dot(h, w3p, preferred_element_type=f32) * dv


def _tc_layer2(s20, s21, s22, s23, dinv, W2, b2, W3, Wp):
    return pl.pallas_call(
        _tc_layer2_body,
        grid=(NP // _BR,),
        in_specs=[pl.BlockSpec((_BR, 128), lambda i: (i, 0))
                  for _ in range(4)] + [
            pl.BlockSpec((_BR, 1), lambda i: (i, 0)),
            pl.BlockSpec((H, H), lambda i: (0, 0)),
            pl.BlockSpec((1, H), lambda i: (0, 0)),
            pl.BlockSpec((H, H), lambda i: (0, 0)),
            pl.BlockSpec((H, 1), lambda i: (0, 0)),
        ],
        out_specs=pl.BlockSpec((_BR, 1), lambda i: (i, 0)),
        out_shape=jax.ShapeDtypeStruct((NP, 1), f32),
    )(s20, s21, s22, s23, dinv, W2, b2, W3, Wp)


def _tc_pool_body(s3a, s3b, dinv_row, batch, b3, Wp, bp, out):
    out3 = (s3a[...] + s3b[...]) * dinv_row[...]
    iota_g = lax.broadcasted_iota(i32, (G, 1), 0)
    maskf = (batch[...] == iota_g).astype(f32)
    counts = jnp.sum(maskf, axis=1, keepdims=True)
    pooled = jnp.sum(maskf * out3, axis=1, keepdims=True)
    b3wp = jnp.dot(b3[...], Wp[...], preferred_element_type=f32)
    out[...] = (pooled / jnp.maximum(counts, 1.0)
                + jnp.where(counts > 0, b3wp, 0.0) + bp[...])


def _tc_pool(s3a, s3b, dinv_row, batch, b3, Wp, bp):
    return pl.pallas_call(
        _tc_pool_body,
        out_shape=jax.ShapeDtypeStruct((G, 1), f32),
    )(s3a, s3b, dinv_row, batch, b3, Wp, bp)


def kernel(x, edge_index, batch, W1, b1, W2, b2, W3, b3, Wp, bp):
    x = x.astype(f32)
    src = edge_index[0]
    dst = edge_index[1]
    src2d = jnp.concatenate(
        [src, jnp.full((EP - E,), PAD_SRC, i32)]).reshape(EP // BATCH, BATCH)
    dst2d = jnp.concatenate(
        [dst, jnp.full((EP - E,), PAD_DST, i32)]).reshape(EP // BATCH, BATCH)
    x_p = jnp.pad(x, ((0, NP - N), (0, 0)))
    batch_row = jnp.pad(batch, (0, NP - N),
                        constant_values=G).reshape(1, NP)
    ones_np = jnp.ones((NP,), f32)
    zeros_np = jnp.zeros((NP,), f32)

    deg0, deg1 = _sc_deg(dst2d, ones_np, zeros_np)
    dinv, dinv_row, xp0, xp1 = _tc_prep(deg0, deg1, x_p)

    s10, s11 = _sc_agg2(xp0, xp1, src2d, dst2d)
    h1p = _tc_layer1(s10, s11, dinv, W1, b1.reshape(1, H))

    s20, s21, s22, s23 = _sc_agg4(*h1p, src2d, dst2d)
    z3p = _tc_layer2(s20, s21, s22, s23, dinv, W2, b2.reshape(1, H), W3, Wp)

    s3a, s3b = _sc_agg_scalar(z3p.reshape(NP), zeros_np, src2d, dst2d)
    out = _tc_pool(s3a.reshape(1, NP), s3b.reshape(1, NP), dinv_row,
                   batch_row, b3.reshape(1, H), Wp, bp.reshape(1, 1))
    return out


# speedup vs baseline: 9.8747x; 9.8747x over previous
import functools

import jax
import jax.numpy as jnp
from jax import lax
from jax.experimental import pallas as pl
from jax.experimental.pallas import tpu as pltpu
from jax.experimental.pallas import tpu_sc as plsc

N = 10000
E = 160000
F_IN = 256
H = 512
G = 64

NP = 10240
EP = 163840
PAD_SRC = N
PAD_DST = N + 16
NC = 2
NS = 16
ROWS_PT = NP // NS
BATCH = 128

f32 = jnp.float32
i32 = jnp.int32

_SC_MESH = dict(mesh=plsc.VectorSubcoreMesh(core_axis_name="c",
                                            subcore_axis_name="s"))


def _sc_deg(dst2d, ones_np, zeros_np):
    nb = EP // (NC * NS) // BATCH

    @functools.partial(
        pl.kernel,
        out_type=(jax.ShapeDtypeStruct((NP,), f32),
                  jax.ShapeDtypeStruct((NP,), f32)),
        scratch_types=[
            pltpu.VMEM((nb, BATCH), i32),
            pltpu.VMEM((BATCH,), f32),
            pltpu.VMEM_SHARED((NP,), f32),
        ],
        **_SC_MESH,
    )
    def k(dst_hbm, ones_hbm, zeros_hbm, out0, out1, idx_v, ones_v, acc):
        c = lax.axis_index("c")
        s = lax.axis_index("s")
        wid = c * NS + s

        @pl.when(c == 0)
        def _():
            pltpu.sync_copy(ones_hbm.at[pl.ds(s * ROWS_PT, ROWS_PT)],
                            acc.at[pl.ds(s * ROWS_PT, ROWS_PT)])

        @pl.when(c == 1)
        def _():
            pltpu.sync_copy(zeros_hbm.at[pl.ds(s * ROWS_PT, ROWS_PT)],
                            acc.at[pl.ds(s * ROWS_PT, ROWS_PT)])

        pltpu.sync_copy(ones_hbm.at[pl.ds(0, BATCH)], ones_v)
        pltpu.sync_copy(dst_hbm.at[pl.ds(wid * nb, nb)], idx_v)
        plsc.subcore_barrier()

        def body(j, carry):
            pltpu.sync_copy(ones_v, acc.at[idx_v.at[j]], add=True)
            return carry

        lax.fori_loop(0, nb, body, 0)
        plsc.subcore_barrier()

        @pl.when(c == 0)
        def _():
            pltpu.sync_copy(acc.at[pl.ds(s * ROWS_PT, ROWS_PT)],
                            out0.at[pl.ds(s * ROWS_PT, ROWS_PT)])

        @pl.when(c == 1)
        def _():
            pltpu.sync_copy(acc.at[pl.ds(s * ROWS_PT, ROWS_PT)],
                            out1.at[pl.ds(s * ROWS_PT, ROWS_PT)])

    return k(dst2d, ones_np, zeros_np)


def _make_sc_agg(cpc):
    nb = EP // NS // BATCH
    nchunks = NC * cpc

    @functools.partial(
        pl.kernel,
        out_type=tuple(jax.ShapeDtypeStruct((NP, 128), f32)
                       for _ in range(nchunks)),
        scratch_types=[
            pltpu.VMEM((nb, BATCH), i32),
            pltpu.VMEM((nb, BATCH), i32),
            pltpu.VMEM((BATCH, 128), f32),
            pltpu.VMEM_SHARED((NP, 128), f32),
            pltpu.SemaphoreType.DMA,
        ],
        **_SC_MESH,
    )
    def k(*args):
        xp = args[:nchunks]
        src2d, dst2d = args[nchunks], args[nchunks + 1]
        outs = args[nchunks + 2:2 * nchunks + 2]
        src_v, dst_v, rows, acc, sem = args[2 * nchunks + 2:]
        c = lax.axis_index("c")
        s = lax.axis_index("s")

        pltpu.sync_copy(src2d.at[pl.ds(s * nb, nb)], src_v)
        pltpu.sync_copy(dst2d.at[pl.ds(s * nb, nb)], dst_v)

        def do_chunk(xp_ref, out_ref):
            pltpu.sync_copy(xp_ref.at[pl.ds(s * ROWS_PT, ROWS_PT)],
                            acc.at[pl.ds(s * ROWS_PT, ROWS_PT)])
            plsc.subcore_barrier()

            def body(j, carry):
                pltpu.async_copy(xp_ref.at[src_v.at[j]], rows, sem).wait()
                pltpu.sync_copy(rows, acc.at[dst_v.at[j]], add=True)
                return carry

            lax.fori_loop(0, nb, body, 0)
            plsc.subcore_barrier()
            pltpu.sync_copy(acc.at[pl.ds(s * ROWS_PT, ROWS_PT)],
                            out_ref.at[pl.ds(s * ROWS_PT, ROWS_PT)])

        for cc in range(NC):
            @pl.when(c == cc)
            def _(cc=cc):
                for q in range(cpc):
                    do_chunk(xp[cc * cpc + q], outs[cc * cpc + q])

    return k


_sc_agg2 = _make_sc_agg(1)
_sc_agg4 = _make_sc_agg(2)


def _sc_agg_scalar(z_hbm_in, zeros_np, src2d, dst2d):
    nb = EP // (NC * NS) // BATCH

    @functools.partial(
        pl.kernel,
        out_type=(jax.ShapeDtypeStruct((NP,), f32),
                  jax.ShapeDtypeStruct((NP,), f32)),
        scratch_types=[
            pltpu.VMEM((nb, BATCH), i32),
            pltpu.VMEM((nb, BATCH), i32),
            pltpu.VMEM((BATCH,), f32),
            pltpu.VMEM_SHARED((NP,), f32),
            pltpu.VMEM_SHARED((NP,), f32),
            pltpu.SemaphoreType.DMA,
        ],
        **_SC_MESH,
    )
    def k(z_hbm, zeros_hbm, src_hbm, dst_hbm, out0, out1,
          src_v, dst_v, vals, z_sp, acc, sem):
        c = lax.axis_index("c")
        s = lax.axis_index("s")
        wid = c * NS + s

        pltpu.sync_copy(z_hbm.at[pl.ds(s * ROWS_PT, ROWS_PT)],
                        z_sp.at[pl.ds(s * ROWS_PT, ROWS_PT)])

        @pl.when(c == 0)
        def _():
            pltpu.sync_copy(z_hbm.at[pl.ds(s * ROWS_PT, ROWS_PT)],
                            acc.at[pl.ds(s * ROWS_PT, ROWS_PT)])

        @pl.when(c == 1)
        def _():
            pltpu.sync_copy(zeros_hbm.at[pl.ds(s * ROWS_PT, ROWS_PT)],
                            acc.at[pl.ds(s * ROWS_PT, ROWS_PT)])

        pltpu.sync_copy(src_hbm.at[pl.ds(wid * nb, nb)], src_v)
        pltpu.sync_copy(dst_hbm.at[pl.ds(wid * nb, nb)], dst_v)
        plsc.subcore_barrier()

        def body(j, carry):
            pltpu.async_copy(z_sp.at[src_v.at[j]], vals, sem).wait()
            pltpu.sync_copy(vals, acc.at[dst_v.at[j]], add=True)
            return carry

        lax.fori_loop(0, nb, body, 0)
        plsc.subcore_barrier()

        @pl.when(c == 0)
        def _():
            pltpu.sync_copy(acc.at[pl.ds(s * ROWS_PT, ROWS_PT)],
                            out0.at[pl.ds(s * ROWS_PT, ROWS_PT)])

        @pl.when(c == 1)
        def _():
            pltpu.sync_copy(acc.at[pl.ds(s * ROWS_PT, ROWS_PT)],
                            out1.at[pl.ds(s * ROWS_PT, ROWS_PT)])

    return k(z_hbm_in, zeros_np, src2d, dst2d)


_BR = 1024


def _tc_prep_body(deg0, deg1, x, dinv, dinv_row, xp0, xp1):
    i = pl.program_id(0)
    rows = i * _BR + lax.broadcasted_iota(i32, (_BR, 1), 0)
    deg = deg0[...] + deg1[...]
    dv = jnp.where(rows < N, lax.rsqrt(deg), 0.0)
    dinv[...] = dv
    dinv_row[...] = dv.reshape(1, _BR)
    xp0[...] = x[:, :128] * dv
    xp1[...] = x[:, 128:] * dv


def _tc_prep(deg0, deg1, x):
    return pl.pallas_call(
        _tc_prep_body,
        grid=(NP // _BR,),
        in_specs=[
            pl.BlockSpec((_BR, 1), lambda i: (i, 0)),
            pl.BlockSpec((_BR, 1), lambda i: (i, 0)),
            pl.BlockSpec((_BR, F_IN), lambda i: (i, 0)),
        ],
        out_specs=[
            pl.BlockSpec((_BR, 1), lambda i: (i, 0)),
            pl.BlockSpec((1, _BR), lambda i: (0, i)),
            pl.BlockSpec((_BR, 128), lambda i: (i, 0)),
            pl.BlockSpec((_BR, 128), lambda i: (i, 0)),
        ],
        out_shape=[
            jax.ShapeDtypeStruct((NP, 1), f32),
            jax.ShapeDtypeStruct((1, NP), f32),
            jax.ShapeDtypeStruct((NP, 128), f32),
            jax.ShapeDtypeStruct((NP, 128), f32),
        ],
    )(deg0, deg1, x)


def _tc_layer1_body(s10, s11, dinv, W1, b1, *outs):
    dv = dinv[...]
    h = (jnp.dot(s10[...] * dv, W1[:128, :], preferred_element_type=f32)
         + jnp.dot(s11[...] * dv, W1[128:, :], preferred_element_type=f32)
         + b1[...])
    h = jnp.maximum(h, 0.0) * dv
    for q in range(4):
        outs[q][...] = h[:, q * 128:(q + 1) * 128]


def _tc_layer1(s10, s11, dinv, W1, b1):
    return pl.pallas_call(
        _tc_layer1_body,
        grid=(NP // _BR,),
        in_specs=[
            pl.BlockSpec((_BR, 128), lambda i: (i, 0)),
            pl.BlockSpec((_BR, 128), lambda i: (i, 0)),
            pl.BlockSpec((_BR, 1), lambda i: (i, 0)),
            pl.BlockSpec((F_IN, H), lambda i: (0, 0)),
            pl.BlockSpec((1, H), lambda i: (0, 0)),
        ],
        out_specs=[pl.BlockSpec((_BR, 128), lambda i: (i, 0))
                   for _ in range(4)],
        out_shape=[jax.ShapeDtypeStruct((NP, 128), f32) for _ in range(4)],
    )(s10, s11, dinv, W1, b1)


def _tc_layer2_body(s20, s21, s22, s23, dinv, W2, b2, W3, Wp, z):
    dv = dinv[...]
    ss = (s20, s21, s22, s23)
    h = b2[...]
    for q in range(4):
        h = h + jnp.dot(ss[q][...] * dv, W2[q * 128:(q + 1) * 128, :],
                        preferred_element_type=f32)
    h = jnp.maximum(h, 0.0)
    w3p = jnp.dot(W3[...], Wp[...], preferred_element_type=f32)
    z[...] = jnp.dot(h, w3p, preferred_element_type=f32) * dv


def _tc_layer2(s20, s21, s22, s23, dinv, W2, b2, W3, Wp):
    return pl.pallas_call(
        _tc_layer2_body,
        grid=(NP // _BR,),
        in_specs=[pl.BlockSpec((_BR, 128), lambda i: (i, 0))
                  for _ in range(4)] + [
            pl.BlockSpec((_BR, 1), lambda i: (i, 0)),
            pl.BlockSpec((H, H), lambda i: (0, 0)),
            pl.BlockSpec((1, H), lambda i: (0, 0)),
            pl.BlockSpec((H, H), lambda i: (0, 0)),
            pl.BlockSpec((H, 1), lambda i: (0, 0)),
        ],
        out_specs=pl.BlockSpec((_BR, 1), lambda i: (i, 0)),
        out_shape=jax.ShapeDtypeStruct((NP, 1), f32),
    )(s20, s21, s22, s23, dinv, W2, b2, W3, Wp)


def _tc_pool_body(s3a, s3b, dinv_row, batch, b3, Wp, bp, out):
    out3 = (s3a[...] + s3b[...]) * dinv_row[...]
    iota_g = lax.broadcasted_iota(i32, (G, 1), 0)
    maskf = (batch[...] == iota_g).astype(f32)
    counts = jnp.sum(maskf, axis=1, keepdims=True)
    pooled = jnp.sum(maskf * out3, axis=1, keepdims=True)
    b3wp = jnp.dot(b3[...], Wp[...], preferred_element_type=f32)
    out[...] = (pooled / jnp.maximum(counts, 1.0)
                + jnp.where(counts > 0, b3wp, 0.0) + bp[...])


def _tc_pool(s3a, s3b, dinv_row, batch, b3, Wp, bp):
    return pl.pallas_call(
        _tc_pool_body,
        out_shape=jax.ShapeDtypeStruct((G, 1), f32),
    )(s3a, s3b, dinv_row, batch, b3, Wp, bp)


def kernel(x, edge_index, batch, W1, b1, W2, b2, W3, b3, Wp, bp):
    x = x.astype(f32)
    src = edge_index[0]
    dst = edge_index[1]
    src2d = jnp.concatenate(
        [src, jnp.full((EP - E,), PAD_SRC, i32)]).reshape(EP // BATCH, BATCH)
    dst2d = jnp.concatenate(
        [dst, jnp.full((EP - E,), PAD_DST, i32)]).reshape(EP // BATCH, BATCH)
    x_p = jnp.pad(x, ((0, NP - N), (0, 0)))
    batch_row = jnp.pad(batch, (0, NP - N),
                        constant_values=G).reshape(1, NP)
    ones_np = jnp.ones((NP,), f32)
    zeros_np = jnp.zeros((NP,), f32)

    deg0, deg1 = _sc_deg(dst2d, ones_np, zeros_np)
    dinv, dinv_row, xp0, xp1 = _tc_prep(deg0.reshape(NP, 1),
                                        deg1.reshape(NP, 1), x_p)

    s10, s11 = _sc_agg2(xp0, xp1, src2d, dst2d)
    h1p = _tc_layer1(s10, s11, dinv, W1, b1.reshape(1, H))

    s20, s21, s22, s23 = _sc_agg4(*h1p, src2d, dst2d)
    z3p = _tc_layer2(s20, s21, s22, s23, dinv, W2, b2.reshape(1, H), W3, Wp)

    s3a, s3b = _sc_agg_scalar(z3p.reshape(NP), zeros_np, src2d, dst2d)
    out = _tc_pool(s3a.reshape(1, NP), s3b.reshape(1, NP), dinv_row,
                   batch_row, b3.reshape(1, H), Wp, bp.reshape(1, 1))
    return out


# speedup vs baseline: 10.4248x; 1.0557x over previous
import functools

import jax
import jax.numpy as jnp
from jax import lax
from jax.experimental import pallas as pl
from jax.experimental.pallas import tpu as pltpu
from jax.experimental.pallas import tpu_sc as plsc

N = 10000
E = 160000
F_IN = 256
H = 512
G = 64

NP = 10240
EP = 163840
PAD_SRC = N
PAD_DST = N + 16
NC = 2
NS = 16
ROWS_PT = NP // NS
BATCH = 128
CHUNK = 128
NBUF = 2
NHALF = 2

f32 = jnp.float32
i32 = jnp.int32

_SC_MESH = dict(mesh=plsc.VectorSubcoreMesh(core_axis_name="c",
                                            subcore_axis_name="s"))


def _sc_deg(dst2d, ones_np, zeros_np):
    nb = EP // (NC * NS) // BATCH

    @functools.partial(
        pl.kernel,
        out_type=(jax.ShapeDtypeStruct((NP,), f32),
                  jax.ShapeDtypeStruct((NP,), f32)),
        scratch_types=[
            pltpu.VMEM((nb, BATCH), i32),
            pltpu.VMEM((BATCH,), f32),
            pltpu.VMEM_SHARED((NP,), f32),
        ],
        **_SC_MESH,
    )
    def k(dst_hbm, ones_hbm, zeros_hbm, out0, out1, idx_v, ones_v, acc):
        c = lax.axis_index("c")
        s = lax.axis_index("s")
        wid = c * NS + s

        @pl.when(c == 0)
        def _():
            pltpu.sync_copy(ones_hbm.at[pl.ds(s * ROWS_PT, ROWS_PT)],
                            acc.at[pl.ds(s * ROWS_PT, ROWS_PT)])

        @pl.when(c == 1)
        def _():
            pltpu.sync_copy(zeros_hbm.at[pl.ds(s * ROWS_PT, ROWS_PT)],
                            acc.at[pl.ds(s * ROWS_PT, ROWS_PT)])

        pltpu.sync_copy(ones_hbm.at[pl.ds(0, BATCH)], ones_v)
        pltpu.sync_copy(dst_hbm.at[pl.ds(wid * nb, nb)], idx_v)
        plsc.subcore_barrier()

        def body(j, carry):
            pltpu.sync_copy(ones_v, acc.at[idx_v.at[j]], add=True)
            return carry

        lax.fori_loop(0, nb, body, 0)
        plsc.subcore_barrier()

        @pl.when(c == 0)
        def _():
            pltpu.sync_copy(acc.at[pl.ds(s * ROWS_PT, ROWS_PT)],
                            out0.at[pl.ds(s * ROWS_PT, ROWS_PT)])

        @pl.when(c == 1)
        def _():
            pltpu.sync_copy(acc.at[pl.ds(s * ROWS_PT, ROWS_PT)],
                            out1.at[pl.ds(s * ROWS_PT, ROWS_PT)])

    return k(dst2d, ones_np, zeros_np)


def _make_sc_agg(cpc):
    nb = EP // NS // BATCH
    nbh = nb // NHALF
    ng = nbh // NBUF
    nchunks = NC * cpc

    @functools.partial(
        pl.kernel,
        out_type=tuple(jax.ShapeDtypeStruct((NP, CHUNK), f32)
                       for _ in range(nchunks)),
        scratch_types=[
            pltpu.VMEM((nbh, BATCH), i32),
            pltpu.VMEM((nbh, BATCH), i32),
            *[pltpu.VMEM((BATCH, CHUNK), f32) for _ in range(NBUF)],
            pltpu.VMEM_SHARED((NP, CHUNK), f32),
            *[pltpu.SemaphoreType.DMA for _ in range(2 * NBUF)],
        ],
        **_SC_MESH,
    )
    def k(*args):
        xp = args[:nchunks]
        src2d, dst2d = args[nchunks], args[nchunks + 1]
        outs = args[nchunks + 2:2 * nchunks + 2]
        rest = args[2 * nchunks + 2:]
        src_v, dst_v = rest[0], rest[1]
        rows = rest[2:2 + NBUF]
        acc = rest[2 + NBUF]
        gsem = rest[3 + NBUF:3 + 2 * NBUF]
        ssem = rest[3 + 2 * NBUF:3 + 3 * NBUF]
        c = lax.axis_index("c")
        s = lax.axis_index("s")

        def do_chunk(xp_ref, out_ref):
            pltpu.sync_copy(xp_ref.at[pl.ds(s * ROWS_PT, ROWS_PT)],
                            acc.at[pl.ds(s * ROWS_PT, ROWS_PT)])
            plsc.subcore_barrier()

            for hh in range(NHALF):
                pltpu.sync_copy(
                    src2d.at[pl.ds(s * nb + hh * nbh, nbh)], src_v)
                pltpu.sync_copy(
                    dst2d.at[pl.ds(s * nb + hh * nbh, nbh)], dst_v)

                def group(g, carry):
                    base = g * NBUF

                    @pl.when(g > 0)
                    def _():
                        for b in range(NBUF):
                            pltpu.make_async_copy(
                                xp_ref.at[pl.ds(0, BATCH)], rows[b],
                                ssem[b]).wait()

                    gds = [pltpu.async_copy(
                        xp_ref.at[src_v.at[base + b]], rows[b], gsem[b])
                        for b in range(NBUF)]
                    for b in range(NBUF):
                        gds[b].wait()
                        pltpu.async_copy(rows[b],
                                         acc.at[dst_v.at[base + b]],
                                         ssem[b], add=True)
                    return carry

                lax.fori_loop(0, ng, group, 0)
                for b in range(NBUF):
                    pltpu.make_async_copy(xp_ref.at[pl.ds(0, BATCH)],
                                          rows[b], ssem[b]).wait()

            plsc.subcore_barrier()
            pltpu.sync_copy(acc.at[pl.ds(s * ROWS_PT, ROWS_PT)],
                            out_ref.at[pl.ds(s * ROWS_PT, ROWS_PT)])

        for cc in range(NC):
            @pl.when(c == cc)
            def _(cc=cc):
                for q in range(cpc):
                    do_chunk(xp[cc * cpc + q], outs[cc * cpc + q])

    return k


_sc_agg2 = _make_sc_agg(1)
_sc_agg4 = _make_sc_agg(2)


def _sc_agg_scalar(z_hbm_in, zeros_np, src2d, dst2d):
    nb = EP // (NC * NS) // BATCH

    @functools.partial(
        pl.kernel,
        out_type=(jax.ShapeDtypeStruct((NP,), f32),
                  jax.ShapeDtypeStruct((NP,), f32)),
        scratch_types=[
            pltpu.VMEM((nb, BATCH), i32),
            pltpu.VMEM((nb, BATCH), i32),
            pltpu.VMEM((BATCH,), f32),
            pltpu.VMEM_SHARED((NP,), f32),
            pltpu.VMEM_SHARED((NP,), f32),
            pltpu.SemaphoreType.DMA,
        ],
        **_SC_MESH,
    )
    def k(z_hbm, zeros_hbm, src_hbm, dst_hbm, out0, out1,
          src_v, dst_v, vals, z_sp, acc, sem):
        c = lax.axis_index("c")
        s = lax.axis_index("s")
        wid = c * NS + s

        pltpu.sync_copy(z_hbm.at[pl.ds(s * ROWS_PT, ROWS_PT)],
                        z_sp.at[pl.ds(s * ROWS_PT, ROWS_PT)])

        @pl.when(c == 0)
        def _():
            pltpu.sync_copy(z_hbm.at[pl.ds(s * ROWS_PT, ROWS_PT)],
                            acc.at[pl.ds(s * ROWS_PT, ROWS_PT)])

        @pl.when(c == 1)
        def _():
            pltpu.sync_copy(zeros_hbm.at[pl.ds(s * ROWS_PT, ROWS_PT)],
                            acc.at[pl.ds(s * ROWS_PT, ROWS_PT)])

        pltpu.sync_copy(src_hbm.at[pl.ds(wid * nb, nb)], src_v)
        pltpu.sync_copy(dst_hbm.at[pl.ds(wid * nb, nb)], dst_v)
        plsc.subcore_barrier()

        def body(j, carry):
            pltpu.async_copy(z_sp.at[src_v.at[j]], vals, sem).wait()
            pltpu.sync_copy(vals, acc.at[dst_v.at[j]], add=True)
            return carry

        lax.fori_loop(0, nb, body, 0)
        plsc.subcore_barrier()

        @pl.when(c == 0)
        def _():
            pltpu.sync_copy(acc.at[pl.ds(s * ROWS_PT, ROWS_PT)],
                            out0.at[pl.ds(s * ROWS_PT, ROWS_PT)])

        @pl.when(c == 1)
        def _():
            pltpu.sync_copy(acc.at[pl.ds(s * ROWS_PT, ROWS_PT)],
                            out1.at[pl.ds(s * ROWS_PT, ROWS_PT)])

    return k(z_hbm_in, zeros_np, src2d, dst2d)


_BR = 1024
NC1 = F_IN // CHUNK
NC2 = H // CHUNK


def _tc_prep_body(deg0, deg1, x, dinv, dinv_row, *xps):
    i = pl.program_id(0)
    rows = i * _BR + lax.broadcasted_iota(i32, (_BR, 1), 0)
    deg = deg0[...] + deg1[...]
    dv = jnp.where(rows < N, lax.rsqrt(deg), 0.0)
    dinv[...] = dv
    dinv_row[...] = dv.reshape(1, _BR)
    xq = x[...] * dv
    for q in range(NC1):
        xps[q][...] = xq[:, q * CHUNK:(q + 1) * CHUNK]


def _tc_prep(deg0, deg1, x):
    return pl.pallas_call(
        _tc_prep_body,
        grid=(NP // _BR,),
        in_specs=[
            pl.BlockSpec((_BR, 1), lambda i: (i, 0)),
            pl.BlockSpec((_BR, 1), lambda i: (i, 0)),
            pl.BlockSpec((_BR, F_IN), lambda i: (i, 0)),
        ],
        out_specs=[
            pl.BlockSpec((_BR, 1), lambda i: (i, 0)),
            pl.BlockSpec((1, _BR), lambda i: (0, i)),
        ] + [pl.BlockSpec((_BR, CHUNK), lambda i: (i, 0))
             for _ in range(NC1)],
        out_shape=[
            jax.ShapeDtypeStruct((NP, 1), f32),
            jax.ShapeDtypeStruct((1, NP), f32),
        ] + [jax.ShapeDtypeStruct((NP, CHUNK), f32) for _ in range(NC1)],
    )(deg0, deg1, x)


def _tc_layer1_body(*refs):
    ss = refs[:NC1]
    dinv, W1, b1 = refs[NC1:NC1 + 3]
    outs = refs[NC1 + 3:]
    dv = dinv[...]
    h = b1[...]
    for q in range(NC1):
        h = h + jnp.dot(ss[q][...] * dv, W1[q * CHUNK:(q + 1) * CHUNK, :],
                        preferred_element_type=f32)
    h = jnp.maximum(h, 0.0) * dv
    for q in range(NC2):
        outs[q][...] = h[:, q * CHUNK:(q + 1) * CHUNK]


def _tc_layer1(ss, dinv, W1, b1):
    return pl.pallas_call(
        _tc_layer1_body,
        grid=(NP // _BR,),
        in_specs=[pl.BlockSpec((_BR, CHUNK), lambda i: (i, 0))
                  for _ in range(NC1)] + [
            pl.BlockSpec((_BR, 1), lambda i: (i, 0)),
            pl.BlockSpec((F_IN, H), lambda i: (0, 0)),
            pl.BlockSpec((1, H), lambda i: (0, 0)),
        ],
        out_specs=[pl.BlockSpec((_BR, CHUNK), lambda i: (i, 0))
                   for _ in range(NC2)],
        out_shape=[jax.ShapeDtypeStruct((NP, CHUNK), f32)
                   for _ in range(NC2)],
    )(*ss, dinv, W1, b1)


def _tc_layer2_body(*refs):
    ss = refs[:NC2]
    dinv, W2, b2, W3, Wp, z = refs[NC2:]
    dv = dinv[...]
    h = b2[...]
    for q in range(NC2):
        h = h + jnp.dot(ss[q][...] * dv, W2[q * CHUNK:(q + 1) * CHUNK, :],
                        preferred_element_type=f32)
    h = jnp.maximum(h, 0.0)
    w3p = jnp.dot(W3[...], Wp[...], preferred_element_type=f32)
    z[...] = jnp.dot(h, w3p, preferred_element_type=f32) * dv


def _tc_layer2(ss, dinv, W2, b2, W3, Wp):
    return pl.pallas_call(
        _tc_layer2_body,
        grid=(NP // _BR,),
        in_specs=[pl.BlockSpec((_BR, CHUNK), lambda i: (i, 0))
                  for _ in range(NC2)] + [
            pl.BlockSpec((_BR, 1), lambda i: (i, 0)),
            pl.BlockSpec((H, H), lambda i: (0, 0)),
            pl.BlockSpec((1, H), lambda i: (0, 0)),
            pl.BlockSpec((H, H), lambda i: (0, 0)),
            pl.BlockSpec((H, 1), lambda i: (0, 0)),
        ],
        out_specs=pl.BlockSpec((_BR, 1), lambda i: (i, 0)),
        out_shape=jax.ShapeDtypeStruct((NP, 1), f32),
    )(*ss, dinv, W2, b2, W3, Wp)


def _tc_pool_body(s3a, s3b, dinv_row, batch, b3, Wp, bp, out):
    out3 = (s3a[...] + s3b[...]) * dinv_row[...]
    iota_g = lax.broadcasted_iota(i32, (G, 1), 0)
    maskf = (batch[...] == iota_g).astype(f32)
    counts = jnp.sum(maskf, axis=1, keepdims=True)
    pooled = jnp.sum(maskf * out3, axis=1, keepdims=True)
    b3wp = jnp.dot(b3[...], Wp[...], preferred_element_type=f32)
    out[...] = (pooled / jnp.maximum(counts, 1.0)
                + jnp.where(counts > 0, b3wp, 0.0) + bp[...])


def _tc_pool(s3a, s3b, dinv_row, batch, b3, Wp, bp):
    return pl.pallas_call(
        _tc_pool_body,
        out_shape=jax.ShapeDtypeStruct((G, 1), f32),
    )(s3a, s3b, dinv_row, batch, b3, Wp, bp)


def kernel(x, edge_index, batch, W1, b1, W2, b2, W3, b3, Wp, bp):
    x = x.astype(f32)
    src = edge_index[0]
    dst = edge_index[1]
    src2d = jnp.concatenate(
        [src, jnp.full((EP - E,), PAD_SRC, i32)]).reshape(EP // BATCH, BATCH)
    dst2d = jnp.concatenate(
        [dst, jnp.full((EP - E,), PAD_DST, i32)]).reshape(EP // BATCH, BATCH)
    x_p = jnp.pad(x, ((0, NP - N), (0, 0)))
    batch_row = jnp.pad(batch, (0, NP - N),
                        constant_values=G).reshape(1, NP)
    ones_np = jnp.ones((NP,), f32)
    zeros_np = jnp.zeros((NP,), f32)

    deg0, deg1 = _sc_deg(dst2d, ones_np, zeros_np)
    prep = _tc_prep(deg0.reshape(NP, 1), deg1.reshape(NP, 1), x_p)
    dinv, dinv_row, xps = prep[0], prep[1], prep[2:]

    s1 = _sc_agg2(*xps, src2d, dst2d)
    h1p = _tc_layer1(s1, dinv, W1, b1.reshape(1, H))

    s2 = _sc_agg4(*h1p, src2d, dst2d)
    z3p = _tc_layer2(s2, dinv, W2, b2.reshape(1, H), W3, Wp)

    s3a, s3b = _sc_agg_scalar(z3p.reshape(NP), zeros_np, src2d, dst2d)
    out = _tc_pool(s3a.reshape(1, NP), s3b.reshape(1, NP), dinv_row,
                   batch_row, b3.reshape(1, H), Wp, bp.reshape(1, 1))
    return out


# speedup vs baseline: 13.3293x; 1.2786x over previous
import functools

import jax
import jax.numpy as jnp
from jax import lax
from jax.experimental import pallas as pl
from jax.experimental.pallas import tpu as pltpu
from jax.experimental.pallas import tpu_sc as plsc

N = 10000
E = 160000
F_IN = 256
H = 512
G = 64

NP = 10240
EP = 163840
PAD_SRC = N
PAD_DST = N + 16
NC = 2
NS = 16
ROWS_PT = NP // NS
BATCH = 128
CHUNK = 64
NBUF = 2
NHALF = 2

f32 = jnp.float32
i32 = jnp.int32

_SC_MESH = dict(mesh=plsc.VectorSubcoreMesh(core_axis_name="c",
                                            subcore_axis_name="s"))


def _sc_deg(dst2d, ones_np, zeros_np):
    nb = EP // (NC * NS) // BATCH

    @functools.partial(
        pl.kernel,
        out_type=(jax.ShapeDtypeStruct((NP,), f32),
                  jax.ShapeDtypeStruct((NP,), f32)),
        scratch_types=[
            pltpu.VMEM((nb, BATCH), i32),
            pltpu.VMEM((BATCH,), f32),
            pltpu.VMEM_SHARED((NP,), f32),
        ],
        **_SC_MESH,
    )
    def k(dst_hbm, ones_hbm, zeros_hbm, out0, out1, idx_v, ones_v, acc):
        c = lax.axis_index("c")
        s = lax.axis_index("s")
        wid = c * NS + s

        @pl.when(c == 0)
        def _():
            pltpu.sync_copy(ones_hbm.at[pl.ds(s * ROWS_PT, ROWS_PT)],
                            acc.at[pl.ds(s * ROWS_PT, ROWS_PT)])

        @pl.when(c == 1)
        def _():
            pltpu.sync_copy(zeros_hbm.at[pl.ds(s * ROWS_PT, ROWS_PT)],
                            acc.at[pl.ds(s * ROWS_PT, ROWS_PT)])

        pltpu.sync_copy(ones_hbm.at[pl.ds(0, BATCH)], ones_v)
        pltpu.sync_copy(dst_hbm.at[pl.ds(wid * nb, nb)], idx_v)
        plsc.subcore_barrier()

        def body(j, carry):
            pltpu.sync_copy(ones_v, acc.at[idx_v.at[j]], add=True)
            return carry

        lax.fori_loop(0, nb, body, 0)
        plsc.subcore_barrier()

        @pl.when(c == 0)
        def _():
            pltpu.sync_copy(acc.at[pl.ds(s * ROWS_PT, ROWS_PT)],
                            out0.at[pl.ds(s * ROWS_PT, ROWS_PT)])

        @pl.when(c == 1)
        def _():
            pltpu.sync_copy(acc.at[pl.ds(s * ROWS_PT, ROWS_PT)],
                            out1.at[pl.ds(s * ROWS_PT, ROWS_PT)])

    return k(dst2d, ones_np, zeros_np)


def _make_sc_agg(cpc):
    nb = EP // NS // BATCH
    nbh = nb // NHALF
    ng = nbh // NBUF
    nchunks = NC * cpc

    @functools.partial(
        pl.kernel,
        out_type=tuple(jax.ShapeDtypeStruct((NP, CHUNK), f32)
                       for _ in range(nchunks)),
        scratch_types=[
            pltpu.VMEM((nbh, BATCH), i32),
            pltpu.VMEM((nbh, BATCH), i32),
            *[pltpu.VMEM((BATCH, CHUNK), f32) for _ in range(NBUF)],
            pltpu.VMEM_SHARED((NP, CHUNK), f32),
            pltpu.VMEM_SHARED((NP, CHUNK), f32),
            *[pltpu.SemaphoreType.DMA for _ in range(2 * NBUF)],
        ],
        compiler_params=pltpu.CompilerParams(use_tc_tiling_on_sc=False),
        **_SC_MESH,
    )
    def k(*args):
        xp = args[:nchunks]
        src2d, dst2d = args[nchunks], args[nchunks + 1]
        outs = args[nchunks + 2:2 * nchunks + 2]
        rest = args[2 * nchunks + 2:]
        src_v, dst_v = rest[0], rest[1]
        rows = rest[2:2 + NBUF]
        src_sp = rest[2 + NBUF]
        acc = rest[3 + NBUF]
        gsem = rest[4 + NBUF:4 + 2 * NBUF]
        ssem = rest[4 + 2 * NBUF:4 + 3 * NBUF]
        c = lax.axis_index("c")
        s = lax.axis_index("s")

        def do_chunk(xp_ref, out_ref):
            pltpu.sync_copy(xp_ref.at[pl.ds(s * ROWS_PT, ROWS_PT)],
                            src_sp.at[pl.ds(s * ROWS_PT, ROWS_PT)])
            pltpu.sync_copy(xp_ref.at[pl.ds(s * ROWS_PT, ROWS_PT)],
                            acc.at[pl.ds(s * ROWS_PT, ROWS_PT)])
            plsc.subcore_barrier()

            for hh in range(NHALF):
                pltpu.sync_copy(
                    src2d.at[pl.ds(s * nb + hh * nbh, nbh)], src_v)
                pltpu.sync_copy(
                    dst2d.at[pl.ds(s * nb + hh * nbh, nbh)], dst_v)

                def group(g, carry):
                    base = g * NBUF

                    @pl.when(g > 0)
                    def _():
                        for b in range(NBUF):
                            pltpu.make_async_copy(
                                xp_ref.at[pl.ds(0, BATCH)], rows[b],
                                ssem[b]).wait()

                    gds = [pltpu.async_copy(
                        src_sp.at[src_v.at[base + b]], rows[b], gsem[b])
                        for b in range(NBUF)]
                    for b in range(NBUF):
                        gds[b].wait()
                        pltpu.async_copy(rows[b],
                                         acc.at[dst_v.at[base + b]],
                                         ssem[b], add=True)
                    return carry

                lax.fori_loop(0, ng, group, 0)
                for b in range(NBUF):
                    pltpu.make_async_copy(xp_ref.at[pl.ds(0, BATCH)],
                                          rows[b], ssem[b]).wait()

            plsc.subcore_barrier()
            pltpu.sync_copy(acc.at[pl.ds(s * ROWS_PT, ROWS_PT)],
                            out_ref.at[pl.ds(s * ROWS_PT, ROWS_PT)])

        for cc in range(NC):
            @pl.when(c == cc)
            def _(cc=cc):
                for q in range(cpc):
                    do_chunk(xp[cc * cpc + q], outs[cc * cpc + q])

    return k


_sc_agg2 = _make_sc_agg(F_IN // CHUNK // NC)
_sc_agg4 = _make_sc_agg(H // CHUNK // NC)


def _sc_agg_scalar(z_hbm_in, zeros_np, src2d, dst2d):
    nb = EP // (NC * NS) // BATCH

    @functools.partial(
        pl.kernel,
        out_type=(jax.ShapeDtypeStruct((NP,), f32),
                  jax.ShapeDtypeStruct((NP,), f32)),
        scratch_types=[
            pltpu.VMEM((nb, BATCH), i32),
            pltpu.VMEM((nb, BATCH), i32),
            pltpu.VMEM((BATCH,), f32),
            pltpu.VMEM_SHARED((NP,), f32),
            pltpu.VMEM_SHARED((NP,), f32),
            pltpu.SemaphoreType.DMA,
        ],
        **_SC_MESH,
    )
    def k(z_hbm, zeros_hbm, src_hbm, dst_hbm, out0, out1,
          src_v, dst_v, vals, z_sp, acc, sem):
        c = lax.axis_index("c")
        s = lax.axis_index("s")
        wid = c * NS + s

        pltpu.sync_copy(z_hbm.at[pl.ds(s * ROWS_PT, ROWS_PT)],
                        z_sp.at[pl.ds(s * ROWS_PT, ROWS_PT)])

        @pl.when(c == 0)
        def _():
            pltpu.sync_copy(z_hbm.at[pl.ds(s * ROWS_PT, ROWS_PT)],
                            acc.at[pl.ds(s * ROWS_PT, ROWS_PT)])

        @pl.when(c == 1)
        def _():
            pltpu.sync_copy(zeros_hbm.at[pl.ds(s * ROWS_PT, ROWS_PT)],
                            acc.at[pl.ds(s * ROWS_PT, ROWS_PT)])

        pltpu.sync_copy(src_hbm.at[pl.ds(wid * nb, nb)], src_v)
        pltpu.sync_copy(dst_hbm.at[pl.ds(wid * nb, nb)], dst_v)
        plsc.subcore_barrier()

        def body(j, carry):
            pltpu.async_copy(z_sp.at[src_v.at[j]], vals, sem).wait()
            pltpu.sync_copy(vals, acc.at[dst_v.at[j]], add=True)
            return carry

        lax.fori_loop(0, nb, body, 0)
        plsc.subcore_barrier()

        @pl.when(c == 0)
        def _():
            pltpu.sync_copy(acc.at[pl.ds(s * ROWS_PT, ROWS_PT)],
                            out0.at[pl.ds(s * ROWS_PT, ROWS_PT)])

        @pl.when(c == 1)
        def _():
            pltpu.sync_copy(acc.at[pl.ds(s * ROWS_PT, ROWS_PT)],
                            out1.at[pl.ds(s * ROWS_PT, ROWS_PT)])

    return k(z_hbm_in, zeros_np, src2d, dst2d)


_BR = 1024
NC1 = F_IN // CHUNK
NC2 = H // CHUNK


def _tc_prep_body(deg0, deg1, x, dinv, dinv_row, *xps):
    i = pl.program_id(0)
    rows = i * _BR + lax.broadcasted_iota(i32, (_BR, 1), 0)
    deg = deg0[...] + deg1[...]
    dv = jnp.where(rows < N, lax.rsqrt(deg), 0.0)
    dinv[...] = dv
    dinv_row[...] = dv.reshape(1, _BR)
    xq = x[...] * dv
    for q in range(NC1):
        xps[q][...] = xq[:, q * CHUNK:(q + 1) * CHUNK]


def _tc_prep(deg0, deg1, x):
    return pl.pallas_call(
        _tc_prep_body,
        grid=(NP // _BR,),
        in_specs=[
            pl.BlockSpec((_BR, 1), lambda i: (i, 0)),
            pl.BlockSpec((_BR, 1), lambda i: (i, 0)),
            pl.BlockSpec((_BR, F_IN), lambda i: (i, 0)),
        ],
        out_specs=[
            pl.BlockSpec((_BR, 1), lambda i: (i, 0)),
            pl.BlockSpec((1, _BR), lambda i: (0, i)),
        ] + [pl.BlockSpec((_BR, CHUNK), lambda i: (i, 0))
             for _ in range(NC1)],
        out_shape=[
            jax.ShapeDtypeStruct((NP, 1), f32),
            jax.ShapeDtypeStruct((1, NP), f32),
        ] + [jax.ShapeDtypeStruct((NP, CHUNK), f32) for _ in range(NC1)],
    )(deg0, deg1, x)


def _tc_layer1_body(*refs):
    ss = refs[:NC1]
    dinv, W1, b1 = refs[NC1:NC1 + 3]
    outs = refs[NC1 + 3:]
    dv = dinv[...]
    h = b1[...]
    for q in range(NC1):
        h = h + jnp.dot(ss[q][...] * dv, W1[q * CHUNK:(q + 1) * CHUNK, :],
                        preferred_element_type=f32)
    h = jnp.maximum(h, 0.0) * dv
    for q in range(NC2):
        outs[q][...] = h[:, q * CHUNK:(q + 1) * CHUNK]


def _tc_layer1(ss, dinv, W1, b1):
    return pl.pallas_call(
        _tc_layer1_body,
        grid=(NP // _BR,),
        in_specs=[pl.BlockSpec((_BR, CHUNK), lambda i: (i, 0))
                  for _ in range(NC1)] + [
            pl.BlockSpec((_BR, 1), lambda i: (i, 0)),
            pl.BlockSpec((F_IN, H), lambda i: (0, 0)),
            pl.BlockSpec((1, H), lambda i: (0, 0)),
        ],
        out_specs=[pl.BlockSpec((_BR, CHUNK), lambda i: (i, 0))
                   for _ in range(NC2)],
        out_shape=[jax.ShapeDtypeStruct((NP, CHUNK), f32)
                   for _ in range(NC2)],
    )(*ss, dinv, W1, b1)


def _tc_layer2_body(*refs):
    ss = refs[:NC2]
    dinv, W2, b2, W3, Wp, z = refs[NC2:]
    dv = dinv[...]
    h = b2[...]
    for q in range(NC2):
        h = h + jnp.dot(ss[q][...] * dv, W2[q * CHUNK:(q + 1) * CHUNK, :],
                        preferred_element_type=f32)
    h = jnp.maximum(h, 0.0)
    w3p = jnp.dot(W3[...], Wp[...], preferred_element_type=f32)
    z[...] = jnp.dot(h, w3p, preferred_element_type=f32) * dv


def _tc_layer2(ss, dinv, W2, b2, W3, Wp):
    return pl.pallas_call(
        _tc_layer2_body,
        grid=(NP // _BR,),
        in_specs=[pl.BlockSpec((_BR, CHUNK), lambda i: (i, 0))
                  for _ in range(NC2)] + [
            pl.BlockSpec((_BR, 1), lambda i: (i, 0)),
            pl.BlockSpec((H, H), lambda i: (0, 0)),
            pl.BlockSpec((1, H), lambda i: (0, 0)),
            pl.BlockSpec((H, H), lambda i: (0, 0)),
            pl.BlockSpec((H, 1), lambda i: (0, 0)),
        ],
        out_specs=pl.BlockSpec((_BR, 1), lambda i: (i, 0)),
        out_shape=jax.ShapeDtypeStruct((NP, 1), f32),
    )(*ss, dinv, W2, b2, W3, Wp)


def _tc_pool_body(s3a, s3b, dinv_row, batch, b3, Wp, bp, out):
    out3 = (s3a[...] + s3b[...]) * dinv_row[...]
    iota_g = lax.broadcasted_iota(i32, (G, 1), 0)
    maskf = (batch[...] == iota_g).astype(f32)
    counts = jnp.sum(maskf, axis=1, keepdims=True)
    pooled = jnp.sum(maskf * out3, axis=1, keepdims=True)
    b3wp = jnp.dot(b3[...], Wp[...], preferred_element_type=f32)
    out[...] = (pooled / jnp.maximum(counts, 1.0)
                + jnp.where(counts > 0, b3wp, 0.0) + bp[...])


def _tc_pool(s3a, s3b, dinv_row, batch, b3, Wp, bp):
    return pl.pallas_call(
        _tc_pool_body,
        out_shape=jax.ShapeDtypeStruct((G, 1), f32),
    )(s3a, s3b, dinv_row, batch, b3, Wp, bp)


def kernel(x, edge_index, batch, W1, b1, W2, b2, W3, b3, Wp, bp):
    x = x.astype(f32)
    src = edge_index[0]
    dst = edge_index[1]
    src2d = jnp.concatenate(
        [src, jnp.full((EP - E,), PAD_SRC, i32)]).reshape(EP // BATCH, BATCH)
    dst2d = jnp.concatenate(
        [dst, jnp.full((EP - E,), PAD_DST, i32)]).reshape(EP // BATCH, BATCH)
    x_p = jnp.pad(x, ((0, NP - N), (0, 0)))
    batch_row = jnp.pad(batch, (0, NP - N),
                        constant_values=G).reshape(1, NP)
    ones_np = jnp.ones((NP,), f32)
    zeros_np = jnp.zeros((NP,), f32)

    deg0, deg1 = _sc_deg(dst2d, ones_np, zeros_np)
    prep = _tc_prep(deg0.reshape(NP, 1), deg1.reshape(NP, 1), x_p)
    dinv, dinv_row, xps = prep[0], prep[1], prep[2:]

    s1 = _sc_agg2(*xps, src2d, dst2d)
    h1p = _tc_layer1(s1, dinv, W1, b1.reshape(1, H))

    s2 = _sc_agg4(*h1p, src2d, dst2d)
    z3p = _tc_layer2(s2, dinv, W2, b2.reshape(1, H), W3, Wp)

    s3a, s3b = _sc_agg_scalar(z3p.reshape(NP), zeros_np, src2d, dst2d)
    out = _tc_pool(s3a.reshape(1, NP), s3b.reshape(1, NP), dinv_row,
                   batch_row, b3.reshape(1, H), Wp, bp.reshape(1, 1))
    return out
